# baseline (speedup 1.0000x reference)
import functools

import jax
import jax.numpy as jnp
from jax import lax
from jax.experimental import pallas as pl
from jax.experimental.pallas import tpu as pltpu
from jax.experimental.pallas import tpu_sc as plsc

_RADIUS2 = 0.2 * 0.2
_K = 32
_PB = 256
_NC = 512
_INT_BIG = 2**30
_D_PAD = 48

_SC_CORES = 2
_SC_SUBCORES = 16
_SC_NW = _SC_CORES * _SC_SUBCORES
_SC_CH = 1024


def _extract_sorted(w, wids, nit, nout):
    PB = w.shape[0]

    def ext_body(k, carry):
        w, rv, ri = carry
        m = jnp.min(w, axis=1)
        sel = w == m[:, None]
        ci = jnp.min(jnp.where(sel, wids, jnp.int32(2**31 - 1)), axis=1)
        onek = lax.broadcasted_iota(jnp.int32, (PB, nout), 1) == k
        rv = jnp.where(onek, m[:, None], rv)
        ri = jnp.where(onek, ci[:, None], ri)
        w = jnp.where(wids == ci[:, None], jnp.inf, w)
        return w, rv, ri

    rv0 = jnp.full((PB, nout), jnp.inf, dtype=jnp.float32)
    ri0 = _INT_BIG + lax.broadcasted_iota(jnp.int32, (PB, nout), 1)
    _, rv, ri = lax.fori_loop(0, nit, ext_body, (w, rv0, ri0))
    return rv, ri


def _ballquery_body(qT_ref, xT_ref, oidx_ref):
    PB = qT_ref.shape[2]
    N = xT_ref.shape[2]
    qb = qT_ref[0]
    q0, q1, q2c = qb[0], qb[1], qb[2]
    qsq = (q0 * q0 + q1 * q1) + q2c * q2c

    runval = jnp.full((PB, _K), jnp.inf, dtype=jnp.float32)
    runidx = _INT_BIG + lax.broadcasted_iota(jnp.int32, (PB, _K), 1)

    for c in range(N // _NC):
        xc = xT_ref[0, :, c * _NC:(c + 1) * _NC]
        x0, x1, x2c = xc[0], xc[1], xc[2]
        xsq = (x0 * x0 + x1 * x1) + x2c * x2c
        dot = lax.dot_general(qb, xc, (((0,), (0,)), ((), ())),
                              preferred_element_type=jnp.float32)
        d2 = jnp.maximum((qsq[:, None] + xsq[None, :]) - 2.0 * dot, 0.0)
        cidx = lax.broadcasted_iota(jnp.int32, (PB, _NC), 1) + c * _NC

        thr = runval[:, _K - 1]
        maskT = d2 < thr[:, None]
        w = jnp.where(maskT, d2, jnp.inf)
        cnt = jnp.sum(maskT.astype(jnp.int32), axis=1)
        m = jnp.minimum(jnp.max(cnt), _K)

        ev, ei = _extract_sorted(w, cidx, m, _K)

        mw = jnp.concatenate([runval, ev], axis=1)
        mi = jnp.concatenate([runidx, ei], axis=1)
        runval, runidx = _extract_sorted(mw, mi, _K, _K)

    mask = runval <= _RADIUS2
    iotaK = lax.broadcasted_iota(jnp.int32, (PB, _K), 1)
    lv = jnp.max(jnp.where(mask, iotaK, -1), axis=1)
    lvc = jnp.maximum(lv, 0)
    gl = jnp.sum(jnp.where(iotaK == lvc[:, None], runidx, 0), axis=1)
    oidx_ref[0] = jnp.where(mask, runidx, gl[:, None])


def _ballquery_idx(qT, xT):
    B, _, P = qT.shape
    N = xT.shape[2]
    return pl.pallas_call(
        _ballquery_body,
        grid=(B, P // _PB),
        in_specs=[
            pl.BlockSpec((1, 3, _PB), lambda b, p: (b, 0, p)),
            pl.BlockSpec((1, 3, N), lambda b, p: (b, 0, 0)),
        ],
        out_specs=pl.BlockSpec((1, _PB, _K), lambda b, p: (b, p, 0)),
        out_shape=jax.ShapeDtypeStruct((B, P, _K), jnp.int32),
        compiler_params=pltpu.CompilerParams(
            dimension_semantics=("parallel", "parallel")),
    )(qT, xT)


def _make_sc_gather(n_idx):
    b_per_w = n_idx // _SC_NW
    mesh = plsc.VectorSubcoreMesh(core_axis_name="c", subcore_axis_name="s")

    @functools.partial(
        pl.kernel, mesh=mesh,
        compiler_params=pltpu.CompilerParams(use_tc_tiling_on_sc=False),
        out_type=jax.ShapeDtypeStruct((n_idx, _D_PAD), jnp.float32),
        scratch_types=[
            pltpu.VMEM((b_per_w,), jnp.int32),
            pltpu.VMEM((_SC_CH, _D_PAD), jnp.float32),
            pltpu.SemaphoreType.DMA,
        ],
    )
    def gather_k(table_hbm, idx_hbm, out_hbm, idx_v, rows_v, sem):
        wid = lax.axis_index("s") * _SC_CORES + lax.axis_index("c")
        base = wid * b_per_w
        pltpu.sync_copy(idx_hbm.at[pl.ds(base, b_per_w)], idx_v)
        for j in range(b_per_w // _SC_CH):
            pltpu.async_copy(
                table_hbm.at[idx_v.at[pl.ds(j * _SC_CH, _SC_CH)]],
                rows_v, sem).wait()
            pltpu.sync_copy(
                rows_v, out_hbm.at[pl.ds(base + j * _SC_CH, _SC_CH)])

    return gather_k


def kernel(xyz, new_xyz, features):
    B, N, _ = xyz.shape
    P = new_xyz.shape[1]
    C = features.shape[1]

    qT = jnp.transpose(new_xyz, (0, 2, 1))
    xT = jnp.transpose(xyz, (0, 2, 1))
    idx = _ballquery_idx(qT, xT)

    table = jnp.concatenate(
        [xyz, jnp.transpose(features, (0, 2, 1)),
         jnp.zeros((B, N, _D_PAD - 3 - C), jnp.float32)], axis=2)
    table = table.reshape(B * N, _D_PAD)
    flat_idx = (idx + (jnp.arange(B, dtype=jnp.int32) * N)[:, None, None])
    flat_idx = flat_idx.reshape(-1)

    g = _make_sc_gather(flat_idx.shape[0])(table, flat_idx)
    g = g.reshape(B, P, _K, _D_PAD)
    gx = g[..., 0:3] - new_xyz[:, :, None, :]
    gf = g[..., 3:3 + C]
    out = jnp.concatenate([gx, gf], axis=-1)
    return jnp.transpose(out, (0, 3, 1, 2))


# speedup vs baseline: 2.6116x; 2.6116x over previous
import functools

import jax
import jax.numpy as jnp
from jax import lax
from jax.experimental import pallas as pl
from jax.experimental.pallas import tpu as pltpu
from jax.experimental.pallas import tpu_sc as plsc

_RADIUS2 = 0.2 * 0.2
_K = 32
_PB = 128
_INT_BIG = 2**30
_D_PAD = 48

_SC_CORES = 2
_SC_SUBCORES = 16
_SC_NW = _SC_CORES * _SC_SUBCORES
_SC_CH = 1024


def _ballquery_body(qT_ref, xT_ref, oidx_ref, w_ref):
    PB = qT_ref.shape[2]
    N = xT_ref.shape[2]
    qb = qT_ref[0]
    xb = xT_ref[0]
    q0, q1, q2c = qb[0], qb[1], qb[2]
    qsq = (q0 * q0 + q1 * q1) + q2c * q2c
    x0, x1, x2c = xb[0], xb[1], xb[2]
    xsq = (x0 * x0 + x1 * x1) + x2c * x2c
    dot = lax.dot_general(qb, xb, (((0,), (0,)), ((), ())),
                          preferred_element_type=jnp.float32)
    w_ref[...] = jnp.maximum((qsq[:, None] + xsq[None, :]) - 2.0 * dot, 0.0)

    def ext_body(k, carry):
        rv, ri = carry
        w = w_ref[...]
        iota = lax.broadcasted_iota(jnp.int32, (PB, N), 1)
        m = jnp.min(w, axis=1)
        ci = jnp.min(jnp.where(w == m[:, None], iota,
                               jnp.int32(2**31 - 1)), axis=1)
        w_ref[...] = jnp.where(iota == ci[:, None], jnp.inf, w)
        onek = lax.broadcasted_iota(jnp.int32, (PB, _K), 1) == k
        rv = jnp.where(onek, m[:, None], rv)
        ri = jnp.where(onek, ci[:, None], ri)
        return rv, ri

    rv0 = jnp.full((PB, _K), jnp.inf, dtype=jnp.float32)
    ri0 = _INT_BIG + lax.broadcasted_iota(jnp.int32, (PB, _K), 1)
    runval, runidx = lax.fori_loop(0, _K, ext_body, (rv0, ri0))

    mask = runval <= _RADIUS2
    iotaK = lax.broadcasted_iota(jnp.int32, (PB, _K), 1)
    lv = jnp.max(jnp.where(mask, iotaK, -1), axis=1)
    lvc = jnp.maximum(lv, 0)
    gl = jnp.sum(jnp.where(iotaK == lvc[:, None], runidx, 0), axis=1)
    oidx_ref[0] = jnp.where(mask, runidx, gl[:, None])


def _ballquery_idx(qT, xT):
    B, _, P = qT.shape
    N = xT.shape[2]
    return pl.pallas_call(
        _ballquery_body,
        grid=(B, P // _PB),
        in_specs=[
            pl.BlockSpec((1, 3, _PB), lambda b, p: (b, 0, p)),
            pl.BlockSpec((1, 3, N), lambda b, p: (b, 0, 0)),
        ],
        out_specs=pl.BlockSpec((1, _PB, _K), lambda b, p: (b, p, 0)),
        out_shape=jax.ShapeDtypeStruct((B, P, _K), jnp.int32),
        scratch_shapes=[pltpu.VMEM((_PB, N), jnp.float32)],
        compiler_params=pltpu.CompilerParams(
            dimension_semantics=("parallel", "parallel")),
    )(qT, xT)


def _make_sc_gather(n_idx):
    b_per_w = n_idx // _SC_NW
    mesh = plsc.VectorSubcoreMesh(core_axis_name="c", subcore_axis_name="s")

    @functools.partial(
        pl.kernel, mesh=mesh,
        compiler_params=pltpu.CompilerParams(use_tc_tiling_on_sc=False),
        out_type=jax.ShapeDtypeStruct((n_idx, _D_PAD), jnp.float32),
        scratch_types=[
            pltpu.VMEM((b_per_w,), jnp.int32),
            pltpu.VMEM((_SC_CH, _D_PAD), jnp.float32),
            pltpu.SemaphoreType.DMA,
        ],
    )
    def gather_k(table_hbm, idx_hbm, out_hbm, idx_v, rows_v, sem):
        wid = lax.axis_index("s") * _SC_CORES + lax.axis_index("c")
        base = wid * b_per_w
        pltpu.sync_copy(idx_hbm.at[pl.ds(base, b_per_w)], idx_v)
        for j in range(b_per_w // _SC_CH):
            pltpu.async_copy(
                table_hbm.at[idx_v.at[pl.ds(j * _SC_CH, _SC_CH)]],
                rows_v, sem).wait()
            pltpu.sync_copy(
                rows_v, out_hbm.at[pl.ds(base + j * _SC_CH, _SC_CH)])

    return gather_k


def kernel(xyz, new_xyz, features):
    B, N, _ = xyz.shape
    P = new_xyz.shape[1]
    C = features.shape[1]

    qT = jnp.transpose(new_xyz, (0, 2, 1))
    xT = jnp.transpose(xyz, (0, 2, 1))
    idx = _ballquery_idx(qT, xT)

    table = jnp.concatenate(
        [xyz, jnp.transpose(features, (0, 2, 1)),
         jnp.zeros((B, N, _D_PAD - 3 - C), jnp.float32)], axis=2)
    table = table.reshape(B * N, _D_PAD)
    flat_idx = (idx + (jnp.arange(B, dtype=jnp.int32) * N)[:, None, None])
    flat_idx = flat_idx.reshape(-1)

    g = _make_sc_gather(flat_idx.shape[0])(table, flat_idx)
    g = g.reshape(B, P, _K, _D_PAD)
    gx = g[..., 0:3] - new_xyz[:, :, None, :]
    gf = g[..., 3:3 + C]
    out = jnp.concatenate([gx, gf], axis=-1)
    return jnp.transpose(out, (0, 3, 1, 2))


# speedup vs baseline: 2.9348x; 1.1237x over previous
import functools

import jax
import jax.numpy as jnp
from jax import lax
from jax.experimental import pallas as pl
from jax.experimental.pallas import tpu as pltpu
from jax.experimental.pallas import tpu_sc as plsc

_RADIUS2 = 0.2 * 0.2
_K = 32
_PB = 128
_INT_BIG = 2**30
_D_PAD = 48

_SC_CORES = 2
_SC_SUBCORES = 16
_SC_NW = _SC_CORES * _SC_SUBCORES
_SC_CH = 1024


def _ballquery_body(qT_ref, xT_ref, oidx_ref, w_ref):
    PB = qT_ref.shape[2]
    N = xT_ref.shape[2]
    qb = qT_ref[0]
    xb = xT_ref[0]
    q0, q1, q2c = qb[0], qb[1], qb[2]
    qsq = (q0 * q0 + q1 * q1) + q2c * q2c
    x0, x1, x2c = xb[0], xb[1], xb[2]
    xsq = (x0 * x0 + x1 * x1) + x2c * x2c
    dot = lax.dot_general(qb, xb, (((0,), (0,)), ((), ())),
                          preferred_element_type=jnp.float32)
    w_ref[...] = jnp.maximum((qsq[:, None] + xsq[None, :]) - 2.0 * dot, 0.0)

    _U = 4

    def ext_body(g, carry):
        rv, ri = carry
        w = w_ref[...]
        iota = lax.broadcasted_iota(jnp.int32, (PB, N), 1)
        iotaK = lax.broadcasted_iota(jnp.int32, (PB, _K), 1)
        for t in range(_U):
            m = jnp.min(w, axis=1)
            ci = jnp.min(jnp.where(w == m[:, None], iota,
                                   jnp.int32(2**31 - 1)), axis=1)
            w = jnp.where(iota == ci[:, None], jnp.inf, w)
            onek = iotaK == g * _U + t
            rv = jnp.where(onek, m[:, None], rv)
            ri = jnp.where(onek, ci[:, None], ri)
        w_ref[...] = w
        return rv, ri

    rv0 = jnp.full((PB, _K), jnp.inf, dtype=jnp.float32)
    ri0 = _INT_BIG + lax.broadcasted_iota(jnp.int32, (PB, _K), 1)
    runval, runidx = lax.fori_loop(0, _K // _U, ext_body, (rv0, ri0))

    mask = runval <= _RADIUS2
    iotaK = lax.broadcasted_iota(jnp.int32, (PB, _K), 1)
    lv = jnp.max(jnp.where(mask, iotaK, -1), axis=1)
    lvc = jnp.maximum(lv, 0)
    gl = jnp.sum(jnp.where(iotaK == lvc[:, None], runidx, 0), axis=1)
    oidx_ref[0] = jnp.where(mask, runidx, gl[:, None])


def _ballquery_idx(qT, xT):
    B, _, P = qT.shape
    N = xT.shape[2]
    return pl.pallas_call(
        _ballquery_body,
        grid=(B, P // _PB),
        in_specs=[
            pl.BlockSpec((1, 3, _PB), lambda b, p: (b, 0, p)),
            pl.BlockSpec((1, 3, N), lambda b, p: (b, 0, 0)),
        ],
        out_specs=pl.BlockSpec((1, _PB, _K), lambda b, p: (b, p, 0)),
        out_shape=jax.ShapeDtypeStruct((B, P, _K), jnp.int32),
        scratch_shapes=[pltpu.VMEM((_PB, N), jnp.float32)],
        compiler_params=pltpu.CompilerParams(
            dimension_semantics=("parallel", "parallel")),
    )(qT, xT)


def _make_sc_gather(n_idx):
    b_per_w = n_idx // _SC_NW
    mesh = plsc.VectorSubcoreMesh(core_axis_name="c", subcore_axis_name="s")

    @functools.partial(
        pl.kernel, mesh=mesh,
        compiler_params=pltpu.CompilerParams(use_tc_tiling_on_sc=False),
        out_type=jax.ShapeDtypeStruct((n_idx, _D_PAD), jnp.float32),
        scratch_types=[
            pltpu.VMEM((b_per_w,), jnp.int32),
            pltpu.VMEM((_SC_CH, _D_PAD), jnp.float32),
            pltpu.SemaphoreType.DMA,
        ],
    )
    def gather_k(table_hbm, idx_hbm, out_hbm, idx_v, rows_v, sem):
        wid = lax.axis_index("s") * _SC_CORES + lax.axis_index("c")
        base = wid * b_per_w
        pltpu.sync_copy(idx_hbm.at[pl.ds(base, b_per_w)], idx_v)
        for j in range(b_per_w // _SC_CH):
            pltpu.async_copy(
                table_hbm.at[idx_v.at[pl.ds(j * _SC_CH, _SC_CH)]],
                rows_v, sem).wait()
            pltpu.sync_copy(
                rows_v, out_hbm.at[pl.ds(base + j * _SC_CH, _SC_CH)])

    return gather_k


def kernel(xyz, new_xyz, features):
    B, N, _ = xyz.shape
    P = new_xyz.shape[1]
    C = features.shape[1]

    qT = jnp.transpose(new_xyz, (0, 2, 1))
    xT = jnp.transpose(xyz, (0, 2, 1))
    idx = _ballquery_idx(qT, xT)

    table = jnp.concatenate(
        [xyz, jnp.transpose(features, (0, 2, 1)),
         jnp.zeros((B, N, _D_PAD - 3 - C), jnp.float32)], axis=2)
    table = table.reshape(B * N, _D_PAD)
    flat_idx = (idx + (jnp.arange(B, dtype=jnp.int32) * N)[:, None, None])
    flat_idx = flat_idx.reshape(-1)

    g = _make_sc_gather(flat_idx.shape[0])(table, flat_idx)
    g = g.reshape(B, P, _K, _D_PAD)
    gx = g[..., 0:3] - new_xyz[:, :, None, :]
    gf = g[..., 3:3 + C]
    out = jnp.concatenate([gx, gf], axis=-1)
    return jnp.transpose(out, (0, 3, 1, 2))


# speedup vs baseline: 2.9733x; 1.0131x over previous
import functools

import jax
import jax.numpy as jnp
from jax import lax
from jax.experimental import pallas as pl
from jax.experimental.pallas import tpu as pltpu
from jax.experimental.pallas import tpu_sc as plsc

_RADIUS2 = 0.2 * 0.2
_K = 32
_PB = 128
_INT_BIG = 2**30
_D_PAD = 48

_SC_CORES = 2
_SC_SUBCORES = 16
_SC_NW = _SC_CORES * _SC_SUBCORES
_SC_CH = 1024


def _ballquery_body(qT_ref, xT_ref, oidx_ref, w_ref, iota_ref):
    PB = qT_ref.shape[2]
    N = xT_ref.shape[2]
    qb = qT_ref[0]
    xb = xT_ref[0]
    q0, q1, q2c = qb[0], qb[1], qb[2]
    qsq = (q0 * q0 + q1 * q1) + q2c * q2c
    x0, x1, x2c = xb[0], xb[1], xb[2]
    xsq = (x0 * x0 + x1 * x1) + x2c * x2c
    dot = lax.dot_general(qb, xb, (((0,), (0,)), ((), ())),
                          preferred_element_type=jnp.float32)
    w_ref[...] = jnp.maximum((qsq[:, None] + xsq[None, :]) - 2.0 * dot, 0.0)
    iota_ref[...] = lax.broadcasted_iota(jnp.int32, (PB, N), 1)

    _U = 8

    def ext_body(g, carry):
        rv, ri = carry
        w = w_ref[...]
        iota = iota_ref[...]
        iotaK = lax.broadcasted_iota(jnp.int32, (PB, _K), 1)
        for t in range(_U):
            m = jnp.min(w, axis=1)
            ci = jnp.min(jnp.where(w == m[:, None], iota,
                                   jnp.int32(2**31 - 1)), axis=1)
            w = jnp.where(iota == ci[:, None], jnp.inf, w)
            onek = iotaK == g * _U + t
            rv = jnp.where(onek, m[:, None], rv)
            ri = jnp.where(onek, ci[:, None], ri)
        w_ref[...] = w
        return rv, ri

    rv0 = jnp.full((PB, _K), jnp.inf, dtype=jnp.float32)
    ri0 = _INT_BIG + lax.broadcasted_iota(jnp.int32, (PB, _K), 1)
    runval, runidx = lax.fori_loop(0, _K // _U, ext_body, (rv0, ri0))

    mask = runval <= _RADIUS2
    iotaK = lax.broadcasted_iota(jnp.int32, (PB, _K), 1)
    lv = jnp.max(jnp.where(mask, iotaK, -1), axis=1)
    lvc = jnp.maximum(lv, 0)
    gl = jnp.sum(jnp.where(iotaK == lvc[:, None], runidx, 0), axis=1)
    oidx_ref[0] = jnp.where(mask, runidx, gl[:, None])


def _ballquery_idx(qT, xT):
    B, _, P = qT.shape
    N = xT.shape[2]
    return pl.pallas_call(
        _ballquery_body,
        grid=(B, P // _PB),
        in_specs=[
            pl.BlockSpec((1, 3, _PB), lambda b, p: (b, 0, p)),
            pl.BlockSpec((1, 3, N), lambda b, p: (b, 0, 0)),
        ],
        out_specs=pl.BlockSpec((1, _PB, _K), lambda b, p: (b, p, 0)),
        out_shape=jax.ShapeDtypeStruct((B, P, _K), jnp.int32),
        scratch_shapes=[pltpu.VMEM((_PB, N), jnp.float32),
                        pltpu.VMEM((_PB, N), jnp.int32)],
        compiler_params=pltpu.CompilerParams(
            dimension_semantics=("parallel", "parallel")),
    )(qT, xT)


def _make_sc_gather(n_idx):
    b_per_w = n_idx // _SC_NW
    mesh = plsc.VectorSubcoreMesh(core_axis_name="c", subcore_axis_name="s")

    @functools.partial(
        pl.kernel, mesh=mesh,
        compiler_params=pltpu.CompilerParams(use_tc_tiling_on_sc=False),
        out_type=jax.ShapeDtypeStruct((n_idx, _D_PAD), jnp.float32),
        scratch_types=[
            pltpu.VMEM((b_per_w,), jnp.int32),
            pltpu.VMEM((_SC_CH, _D_PAD), jnp.float32),
            pltpu.SemaphoreType.DMA,
        ],
    )
    def gather_k(table_hbm, idx_hbm, out_hbm, idx_v, rows_v, sem):
        wid = lax.axis_index("s") * _SC_CORES + lax.axis_index("c")
        base = wid * b_per_w
        pltpu.sync_copy(idx_hbm.at[pl.ds(base, b_per_w)], idx_v)
        for j in range(b_per_w // _SC_CH):
            pltpu.async_copy(
                table_hbm.at[idx_v.at[pl.ds(j * _SC_CH, _SC_CH)]],
                rows_v, sem).wait()
            pltpu.sync_copy(
                rows_v, out_hbm.at[pl.ds(base + j * _SC_CH, _SC_CH)])

    return gather_k


def kernel(xyz, new_xyz, features):
    B, N, _ = xyz.shape
    P = new_xyz.shape[1]
    C = features.shape[1]

    qT = jnp.transpose(new_xyz, (0, 2, 1))
    xT = jnp.transpose(xyz, (0, 2, 1))
    idx = _ballquery_idx(qT, xT)

    table = jnp.concatenate(
        [xyz, jnp.transpose(features, (0, 2, 1)),
         jnp.zeros((B, N, _D_PAD - 3 - C), jnp.float32)], axis=2)
    table = table.reshape(B * N, _D_PAD)
    flat_idx = (idx + (jnp.arange(B, dtype=jnp.int32) * N)[:, None, None])
    flat_idx = flat_idx.reshape(-1)

    g = _make_sc_gather(flat_idx.shape[0])(table, flat_idx)
    g = g.reshape(B, P, _K, _D_PAD)
    gx = g[..., 0:3] - new_xyz[:, :, None, :]
    gf = g[..., 3:3 + C]
    out = jnp.concatenate([gx, gf], axis=-1)
    return jnp.transpose(out, (0, 3, 1, 2))
